# baseline (speedup 1.0000x reference)
import functools

import jax
import jax.numpy as jnp
from jax import lax
from jax.experimental import pallas as pl
from jax.experimental.pallas import tpu as pltpu
from jax.experimental.pallas import tpu_sc as plsc

_N, _L, _C = 2, 4096, 1024
_K, _G = 7, 4
_GC = _C // _G
_SCALE = 2.0
_LB = 1024
_NI = _L // _LB
_SQRT_HALF = 0.7071067811865476

_NC, _NS = 2, 16
_NW = _NC * _NS
_LW = (_N * _L) // _NW
_NT = _LW // 16


def _prep_body(xp, xc, xn, dww, dwb, lng, lnb, inw, inb, omw, omb,
               proj_ref, om_ref):
    i = pl.program_id(1)
    x = xc[0]
    left = jnp.where(i > 0, xp[0, _LB - 1:_LB, :], 0.0)
    right = jnp.where(i < _NI - 1, xn[0, 0:1, :], 0.0)
    xm1 = jnp.concatenate([left, x[:-1]], axis=0)
    xp1 = jnp.concatenate([x[1:], right], axis=0)
    xdw = xm1 * dww[0:1] + x * dww[1:2] + xp1 * dww[2:3] + dwb[...]
    mu = jnp.mean(xdw, axis=-1, keepdims=True)
    xz = xdw - mu
    var = jnp.mean(xz * xz, axis=-1, keepdims=True)
    xdw = xz * lax.rsqrt(var + 1e-5) * lng[...] + lnb[...]
    xdw = 0.5 * xdw * (1.0 + lax.erf(xdw * _SQRT_HALF))
    proj = (lax.dot_general(x, inw[...], (((1,), (1,)), ((), ())),
                            preferred_element_type=jnp.float32) + inb[...])
    proj_ref[0] = jnp.transpose(proj.reshape(_LB, _G, _GC), (1, 0, 2))
    om_ref[0] = (
        lax.dot_general(omw[...], xdw, (((1,), (1,)), ((), ())),
                        preferred_element_type=jnp.float32) + omb[...])


def _sc_gather_body(table_hbm, om_hbm, out_hbm, om_v, buf, outbuf, gsem, osem):
    wid = lax.axis_index("s") * _NC + lax.axis_index("c")
    n = wid // _NS
    l0 = (wid % _NS) * _LW
    pltpu.sync_copy(om_hbm.at[n, :, pl.ds(l0, _LW)], om_v)

    rowiota = lax.iota(jnp.int32, 16)
    fiota = rowiota.astype(jnp.float32)

    def chunk_body(t, carry):
        lbase = l0 + t * 16
        lvec = lbase.astype(jnp.float32) + fiota
        for g in range(_G):
            base_row = (n * _G + g) * _L
            mraw = [om_v[_G * _K + g * _K + k, pl.ds(t * 16, 16)]
                    for k in range(_K)]
            gmax = mraw[0]
            for k in range(1, _K):
                gmax = jnp.maximum(gmax, mraw[k])
            mexp = [jnp.exp(mr - gmax) for mr in mraw]
            msum = mexp[0]
            for k in range(1, _K):
                msum = msum + mexp[k]
            rinv = 1.0 / msum

            idx_list, w_list = [], []
            for k in range(_K):
                off = om_v[g * _K + k, pl.ds(t * 16, 16)] * _SCALE
                abs_pos = lvec + (k - _K // 2) + off
                apc = jnp.clip(abs_pos, 0.0, float(_L - 1))
                pf = apc.astype(jnp.int32)
                pc = jnp.minimum(pf + 1, _L - 1)
                wc = apc - pf.astype(jnp.float32)
                wf = 1.0 - wc
                valid = jnp.logical_and(abs_pos >= 0.0,
                                        abs_pos <= float(_L - 1))
                vm = jnp.where(valid, mexp[k], 0.0)
                w_list.append(wf * vm)
                w_list.append(wc * vm)
                idx_list.append(base_row + pf)
                idx_list.append(base_row + pc)

            cps = [pltpu.async_copy(table_hbm.at[idx_list[j]], buf.at[j], gsem)
                   for j in range(2 * _K)]
            for cp in cps:
                cp.wait()

            def dbody(d, c):
                dvec = jnp.full((16,), d, jnp.int32)
                acc = jnp.zeros((16,), jnp.float32)
                for j in range(2 * _K):
                    jvec = jnp.full((16,), j, jnp.int32)
                    v = plsc.load_gather(buf, [jvec, rowiota, dvec])
                    acc = acc + w_list[j] * v
                plsc.store_scatter(outbuf, [rowiota, dvec], acc * rinv)
                return c

            lax.fori_loop(0, _GC, dbody, 0)
            pltpu.async_copy(
                outbuf, out_hbm.at[pl.ds(base_row + lbase, 16)], osem
            ).wait()
        return carry

    lax.fori_loop(0, _NT, chunk_body, 0)


def _sc_gather(table, om):
    mesh = plsc.VectorSubcoreMesh(core_axis_name="c", subcore_axis_name="s")
    f = functools.partial(
        pl.kernel,
        out_type=jax.ShapeDtypeStruct((_N * _G * _L, _GC), jnp.float32),
        mesh=mesh,
        compiler_params=pltpu.CompilerParams(
            use_tc_tiling_on_sc=False, needs_layout_passes=False),
        scratch_types=[
            pltpu.VMEM((2 * _G * _K, _LW), jnp.float32),
            pltpu.VMEM((2 * _K, 16, _GC), jnp.float32),
            pltpu.VMEM((16, _GC), jnp.float32),
            pltpu.SemaphoreType.DMA,
            pltpu.SemaphoreType.DMA,
        ],
    )(_sc_gather_body)
    return f(table, om)


def _outproj_body(y, w, b, o_ref):
    g = pl.program_id(2)
    part = lax.dot_general(y[0, 0], w[0], (((1,), (0,)), ((), ())),
                           preferred_element_type=jnp.float32)

    @pl.when(g == 0)
    def _():
        o_ref[0] = part + b[...]

    @pl.when(g > 0)
    def _():
        o_ref[0] += part


def kernel(x, dw_w, dw_b, ln_g, ln_b, off_w, off_b, mask_w, mask_b,
           in_w, in_b, out_w, out_b):
    n, l, c = x.shape
    dww = jnp.transpose(dw_w[:, 0, :])
    omw = jnp.concatenate([off_w, mask_w], axis=0)
    omb = jnp.concatenate([off_b, mask_b], axis=0)[:, None]

    proj, om = pl.pallas_call(
        _prep_body,
        grid=(_N, _NI),
        in_specs=[
            pl.BlockSpec((1, _LB, _C), lambda n_, i: (n_, jnp.maximum(i - 1, 0), 0)),
            pl.BlockSpec((1, _LB, _C), lambda n_, i: (n_, i, 0)),
            pl.BlockSpec((1, _LB, _C), lambda n_, i: (n_, jnp.minimum(i + 1, _NI - 1), 0)),
            pl.BlockSpec((3, _C), lambda n_, i: (0, 0)),
            pl.BlockSpec((1, _C), lambda n_, i: (0, 0)),
            pl.BlockSpec((1, _C), lambda n_, i: (0, 0)),
            pl.BlockSpec((1, _C), lambda n_, i: (0, 0)),
            pl.BlockSpec((_C, _C), lambda n_, i: (0, 0)),
            pl.BlockSpec((1, _C), lambda n_, i: (0, 0)),
            pl.BlockSpec((_G * _K * 2, _C), lambda n_, i: (0, 0)),
            pl.BlockSpec((_G * _K * 2, 1), lambda n_, i: (0, 0)),
        ],
        out_specs=[
            pl.BlockSpec((1, _G, _LB, _GC), lambda n_, i: (n_, 0, i, 0)),
            pl.BlockSpec((1, _G * _K * 2, _LB), lambda n_, i: (n_, 0, i)),
        ],
        out_shape=[
            jax.ShapeDtypeStruct((_N, _G, _L, _GC), jnp.float32),
            jax.ShapeDtypeStruct((_N, _G * _K * 2, _L), jnp.float32),
        ],
    )(x, x, x, dww, dw_b[None], ln_g[None], ln_b[None], in_w, in_b[None],
      omw, omb)

    y = _sc_gather(proj.reshape(_N * _G * _L, _GC), om)
    y = y.reshape(_N, _G, _L, _GC)

    w_r = jnp.transpose(out_w).reshape(_G, _GC, _C)
    o = pl.pallas_call(
        _outproj_body,
        grid=(_N, _NI, _G),
        in_specs=[
            pl.BlockSpec((1, 1, _LB, _GC), lambda n_, i, g: (n_, g, i, 0)),
            pl.BlockSpec((1, _GC, _C), lambda n_, i, g: (g, 0, 0)),
            pl.BlockSpec((1, _C), lambda n_, i, g: (0, 0)),
        ],
        out_specs=pl.BlockSpec((1, _LB, _C), lambda n_, i, g: (n_, i, 0)),
        out_shape=jax.ShapeDtypeStruct((_N, _L, _C), jnp.float32),
    )(y, w_r, out_b[None])
    return o


# speedup vs baseline: 1.7521x; 1.7521x over previous
import functools

import jax
import jax.numpy as jnp
from jax import lax
from jax.experimental import pallas as pl
from jax.experimental.pallas import tpu as pltpu
from jax.experimental.pallas import tpu_sc as plsc

_N, _L, _C = 2, 4096, 1024
_K, _G = 7, 4
_GC = _C // _G
_GC2 = _GC // 2
_SCALE = 2.0
_LB = 1024
_NI = _L // _LB
_LBP = 512
_NIP = _L // _LBP
_SQRT_HALF = 0.7071067811865476

_NC, _NS = 2, 16
_NW = _NC * _NS
_LW = (_N * _L) // _NW
_NT = _LW // 16
_NSTEP = _NT * _G
_J = 2 * _K


def _prep_body(xp, xc, xn, dww, dwb, lng, lnb, inw, inb, omw, omb,
               proj_ref, om_ref):
    i = pl.program_id(1)
    x = xc[0]
    left = jnp.where(i > 0, xp[0, _LBP - 1:_LBP, :], 0.0)
    right = jnp.where(i < _NIP - 1, xn[0, 0:1, :], 0.0)
    xm1 = jnp.concatenate([left, x[:-1]], axis=0)
    xp1 = jnp.concatenate([x[1:], right], axis=0)
    xdw = xm1 * dww[0:1] + x * dww[1:2] + xp1 * dww[2:3] + dwb[...]
    mu = jnp.mean(xdw, axis=-1, keepdims=True)
    xz = xdw - mu
    var = jnp.mean(xz * xz, axis=-1, keepdims=True)
    xdw = xz * lax.rsqrt(var + 1e-5) * lng[...] + lnb[...]
    xdw = 0.5 * xdw * (1.0 + lax.erf(xdw * _SQRT_HALF))
    proj = (lax.dot_general(x, inw[...], (((1,), (1,)), ((), ())),
                            preferred_element_type=jnp.float32) + inb[...])
    proj_ref[0] = jnp.transpose(proj.reshape(_LBP, _G, _GC), (1, 0, 2)).astype(
        jnp.bfloat16)
    om_ref[0] = (
        lax.dot_general(omw[...], xdw, (((1,), (1,)), ((), ())),
                        preferred_element_type=jnp.float32) + omb[...])


_HB = _J * 16 // 2


def _sc_body(table_hbm, om_hbm, out_hbm, om_v, idxb, buf, outbuf,
             gs0, gs1, os0, os1):
    wid = lax.axis_index("s") * _NC + lax.axis_index("c")
    n = wid // _NS
    l0 = (wid % _NS) * _LW
    pltpu.sync_copy(om_hbm.at[n, :, pl.ds(l0, _LW)], om_v)

    rowiota = lax.iota(jnp.int32, 16)
    fiota = rowiota.astype(jnp.float32)

    def calc(s):
        t = s // _G
        g = s % _G
        lbase = l0 + t * 16
        lvec = lax.convert_element_type(lbase, jnp.float32) + fiota
        base_row = (n * _G + g) * _L
        mraw = [om_v[_G * _K + g * _K + k, pl.ds(t * 16, 16)]
                for k in range(_K)]
        gmax = mraw[0]
        for k in range(1, _K):
            gmax = jnp.maximum(gmax, mraw[k])
        mexp = [jnp.exp(mr - gmax) for mr in mraw]
        msum = mexp[0]
        for k in range(1, _K):
            msum = msum + mexp[k]
        rinv = 1.0 / msum
        idx_list, w_list = [], []
        for k in range(_K):
            off = om_v[g * _K + k, pl.ds(t * 16, 16)] * _SCALE
            abs_pos = lvec + (k - _K // 2) + off
            apc = jnp.clip(abs_pos, 0.0, float(_L - 1))
            pf = apc.astype(jnp.int32)
            pc = jnp.minimum(pf + 1, _L - 1)
            wc = apc - pf.astype(jnp.float32)
            wf = 1.0 - wc
            valid = jnp.logical_and(abs_pos >= 0.0, abs_pos <= float(_L - 1))
            vm = jnp.where(valid, mexp[k], 0.0)
            w_list.append(wf * vm)
            w_list.append(wc * vm)
            idx_list.append(base_row + pf)
            idx_list.append(base_row + pc)
        return idx_list, tuple(w_list) + (rinv,)

    def fire(idx_list, slot):
        sem = gs0 if slot == 0 else gs1
        for j in range(_J):
            idxb[slot, j // _K, pl.ds((j % _K) * 16, 16)] = idx_list[j]
        for h in range(2):
            pltpu.async_copy(
                table_hbm.at[idxb.at[slot, h]],
                buf.at[pl.ds(slot * 2 * _HB + h * _HB, _HB)], sem)

    def drain(slot):
        sem = gs0 if slot == 0 else gs1
        for h in range(2):
            pltpu.make_async_copy(
                table_hbm.at[idxb.at[slot, h]],
                buf.at[pl.ds(slot * 2 * _HB + h * _HB, _HB)], sem).wait()

    idx0, w0 = calc(0)
    fire(idx0, 0)

    def body(s, wcur):
        p = s % 2
        @pl.when(jnp.logical_and(s >= 2, p == 0))
        def _():
            pltpu.make_async_copy(
                outbuf.at[pl.ds(0, 16)], out_hbm.at[pl.ds(0, 16)], os0).wait()

        @pl.when(jnp.logical_and(s >= 2, p == 1))
        def _():
            pltpu.make_async_copy(
                outbuf.at[pl.ds(16, 16)], out_hbm.at[pl.ds(0, 16)], os1).wait()

        s1 = jnp.minimum(s + 1, _NSTEP - 1)
        idxn, wnext = calc(s1)
        not_last = s < _NSTEP - 1

        @pl.when(jnp.logical_and(not_last, p == 0))
        def _():
            fire(idxn, 1)

        @pl.when(jnp.logical_and(not_last, p == 1))
        def _():
            fire(idxn, 0)

        @pl.when(p == 0)
        def _():
            drain(0)

        @pl.when(p == 1)
        def _():
            drain(1)

        rinv = wcur[_J]
        rbase = p * (2 * _HB) + rowiota
        rvecs = [rbase + j * 16 for j in range(_J)]
        obase = p * 16 + rowiota

        def dbody(d2, c):
            d2vec = jnp.full((16,), d2, jnp.int32)
            acc0 = jnp.zeros((16,), jnp.float32)
            acc1 = jnp.zeros((16,), jnp.float32)
            for j in range(_J):
                wrd = plsc.load_gather(buf, [rvecs[j], d2vec])
                ev = plsc.bitcast(lax.shift_left(wrd, 16), jnp.float32)
                od = plsc.bitcast(
                    lax.bitwise_and(wrd, jnp.int32(-65536)), jnp.float32)
                acc0 = acc0 + wcur[j] * ev
                acc1 = acc1 + wcur[j] * od
            plsc.store_scatter(outbuf, [obase, d2vec * 2], acc0 * rinv)
            plsc.store_scatter(outbuf, [obase, d2vec * 2 + 1], acc1 * rinv)
            return c

        lax.fori_loop(0, _GC2, dbody, 0)

        t = s // _G
        g = s % _G
        orow = (n * _G + g) * _L + l0 + t * 16

        @pl.when(p == 0)
        def _():
            pltpu.async_copy(
                outbuf.at[pl.ds(0, 16)], out_hbm.at[pl.ds(orow, 16)], os0)

        @pl.when(p == 1)
        def _():
            pltpu.async_copy(
                outbuf.at[pl.ds(16, 16)], out_hbm.at[pl.ds(orow, 16)], os1)

        return wnext

    lax.fori_loop(0, _NSTEP, body, w0)
    pltpu.make_async_copy(
        outbuf.at[pl.ds(0, 16)], out_hbm.at[pl.ds(0, 16)], os0).wait()
    pltpu.make_async_copy(
        outbuf.at[pl.ds(16, 16)], out_hbm.at[pl.ds(0, 16)], os1).wait()


def _sc_gather(table_i32, om):
    mesh = plsc.VectorSubcoreMesh(core_axis_name="c", subcore_axis_name="s")
    f = functools.partial(
        pl.kernel,
        out_type=jax.ShapeDtypeStruct((_N * _G * _L, _GC), jnp.float32),
        mesh=mesh,
        compiler_params=pltpu.CompilerParams(
            use_tc_tiling_on_sc=False, needs_layout_passes=False),
        scratch_types=[
            pltpu.VMEM((2 * _G * _K, _LW), jnp.float32),
            pltpu.VMEM((2, 2, _HB), jnp.int32),
            pltpu.VMEM((4 * _HB, _GC2), jnp.int32),
            pltpu.VMEM((32, _GC), jnp.float32),
            pltpu.SemaphoreType.DMA,
            pltpu.SemaphoreType.DMA,
            pltpu.SemaphoreType.DMA,
            pltpu.SemaphoreType.DMA,
        ],
    )(_sc_body)
    return f(table_i32, om)


def _outproj_body(y, w, b, o_ref):
    g = pl.program_id(2)
    part = lax.dot_general(y[0, 0], w[0], (((1,), (0,)), ((), ())),
                           preferred_element_type=jnp.float32)

    @pl.when(g == 0)
    def _():
        o_ref[0] = part + b[...]

    @pl.when(g > 0)
    def _():
        o_ref[0] += part


def kernel(x, dw_w, dw_b, ln_g, ln_b, off_w, off_b, mask_w, mask_b,
           in_w, in_b, out_w, out_b):
    n, l, c = x.shape
    dww = jnp.transpose(dw_w[:, 0, :])
    omw = jnp.concatenate([off_w, mask_w], axis=0)
    omb = jnp.concatenate([off_b, mask_b], axis=0)[:, None]

    proj, om = pl.pallas_call(
        _prep_body,
        grid=(_N, _NIP),
        in_specs=[
            pl.BlockSpec((1, _LBP, _C), lambda n_, i: (n_, jnp.maximum(i - 1, 0), 0)),
            pl.BlockSpec((1, _LBP, _C), lambda n_, i: (n_, i, 0)),
            pl.BlockSpec((1, _LBP, _C), lambda n_, i: (n_, jnp.minimum(i + 1, _NIP - 1), 0)),
            pl.BlockSpec((3, _C), lambda n_, i: (0, 0)),
            pl.BlockSpec((1, _C), lambda n_, i: (0, 0)),
            pl.BlockSpec((1, _C), lambda n_, i: (0, 0)),
            pl.BlockSpec((1, _C), lambda n_, i: (0, 0)),
            pl.BlockSpec((_C, _C), lambda n_, i: (0, 0)),
            pl.BlockSpec((1, _C), lambda n_, i: (0, 0)),
            pl.BlockSpec((_G * _K * 2, _C), lambda n_, i: (0, 0)),
            pl.BlockSpec((_G * _K * 2, 1), lambda n_, i: (0, 0)),
        ],
        out_specs=[
            pl.BlockSpec((1, _G, _LBP, _GC), lambda n_, i: (n_, 0, i, 0)),
            pl.BlockSpec((1, _G * _K * 2, _LBP), lambda n_, i: (n_, 0, i)),
        ],
        out_shape=[
            jax.ShapeDtypeStruct((_N, _G, _L, _GC), jnp.bfloat16),
            jax.ShapeDtypeStruct((_N, _G * _K * 2, _L), jnp.float32),
        ],
    )(x, x, x, dww, dw_b[None], ln_g[None], ln_b[None], in_w, in_b[None],
      omw, omb)

    table_i32 = lax.bitcast_convert_type(
        proj.reshape(_N * _G * _L, _GC2, 2), jnp.int32)
    y = _sc_gather(table_i32, om)
    y = y.reshape(_N, _G, _L, _GC)

    w_r = jnp.transpose(out_w).reshape(_G, _GC, _C)
    o = pl.pallas_call(
        _outproj_body,
        grid=(_N, _NI, _G),
        in_specs=[
            pl.BlockSpec((1, 1, _LB, _GC), lambda n_, i, g: (n_, g, i, 0)),
            pl.BlockSpec((1, _GC, _C), lambda n_, i, g: (g, 0, 0)),
            pl.BlockSpec((1, _C), lambda n_, i, g: (0, 0)),
        ],
        out_specs=pl.BlockSpec((1, _LB, _C), lambda n_, i, g: (n_, i, 0)),
        out_shape=jax.ShapeDtypeStruct((_N, _L, _C), jnp.float32),
    )(y, w_r, out_b[None])
    return o


# speedup vs baseline: 5.0260x; 2.8685x over previous
import functools

import numpy as np

import jax
import jax.numpy as jnp
from jax import lax
from jax.experimental import pallas as pl
from jax.experimental.pallas import tpu as pltpu
from jax.experimental.pallas import tpu_sc as plsc

_N, _L, _C = 2, 4096, 1024
_K, _G = 7, 4
_GC = _C // _G
_GC2 = _GC // 2
_SCALE = 2.0
_LB = 1024
_NI = _L // _LB
_LBP = 512
_NIP = _L // _LBP
_SQRT_HALF = 0.7071067811865476

_NC, _NS = 2, 16
_NW = _NC * _NS
_LW = (_N * _L) // _NW
_NT = _LW // 16
_NSTEP = _NT * _G
_J = 2 * _K


def _prep_body(xp, xc, xn, dww, dwb, lng, lnb, inw, inb, omw, omb,
               proj_ref, om_ref):
    i = pl.program_id(1)
    x = xc[0]
    left = jnp.where(i > 0, xp[0, _LBP - 1:_LBP, :], 0.0)
    right = jnp.where(i < _NIP - 1, xn[0, 0:1, :], 0.0)
    xm1 = jnp.concatenate([left, x[:-1]], axis=0)
    xp1 = jnp.concatenate([x[1:], right], axis=0)
    xdw = xm1 * dww[0:1] + x * dww[1:2] + xp1 * dww[2:3] + dwb[...]
    mu = jnp.mean(xdw, axis=-1, keepdims=True)
    xz = xdw - mu
    var = jnp.mean(xz * xz, axis=-1, keepdims=True)
    xdw = xz * lax.rsqrt(var + 1e-5) * lng[...] + lnb[...]
    xdw = 0.5 * xdw * (1.0 + lax.erf(xdw * _SQRT_HALF))
    proj = (lax.dot_general(x, inw[...], (((1,), (1,)), ((), ())),
                            preferred_element_type=jnp.float32) + inb[...])
    proj_ref[0] = jnp.transpose(proj.reshape(_LBP, _G, _GC), (1, 0, 2)).astype(
        jnp.bfloat16)
    om_ref[0] = (
        lax.dot_general(omw[...], xdw, (((1,), (1,)), ((), ())),
                        preferred_element_type=jnp.float32) + omb[...])


_HB = _J * 16 // 2


def _sc_body(table_hbm, om_hbm, out_hbm, om_v, idxb, buf, outbuf,
             gs0, gs1, os0, os1):
    wid = lax.axis_index("s") * _NC + lax.axis_index("c")
    n = wid // _NS
    l0 = (wid % _NS) * _LW
    pltpu.sync_copy(om_hbm.at[n, :, pl.ds(l0, _LW)], om_v)

    rowiota = lax.iota(jnp.int32, 16)
    fiota = rowiota.astype(jnp.float32)

    def calc(s):
        t = s // _G
        g = s % _G
        lbase = l0 + t * 16
        lvec = lax.convert_element_type(lbase, jnp.float32) + fiota
        base_row = (n * _G + g) * _L
        mraw = [om_v[_G * _K + g * _K + k, pl.ds(t * 16, 16)]
                for k in range(_K)]
        gmax = mraw[0]
        for k in range(1, _K):
            gmax = jnp.maximum(gmax, mraw[k])
        mexp = [jnp.exp(mr - gmax) for mr in mraw]
        msum = mexp[0]
        for k in range(1, _K):
            msum = msum + mexp[k]
        rinv = 1.0 / msum
        idx_list, w_list = [], []
        for k in range(_K):
            off = om_v[g * _K + k, pl.ds(t * 16, 16)] * _SCALE
            abs_pos = lvec + (k - _K // 2) + off
            apc = jnp.clip(abs_pos, 0.0, float(_L - 1))
            pf = apc.astype(jnp.int32)
            pc = jnp.minimum(pf + 1, _L - 1)
            wc = apc - pf.astype(jnp.float32)
            wf = 1.0 - wc
            valid = jnp.logical_and(abs_pos >= 0.0, abs_pos <= float(_L - 1))
            vm = jnp.where(valid, mexp[k], 0.0)
            w_list.append(wf * vm)
            w_list.append(wc * vm)
            idx_list.append(base_row + pf)
            idx_list.append(base_row + pc)
        return idx_list, tuple(w_list) + (rinv,)

    def fire(idx_list, slot):
        sem = gs0 if slot == 0 else gs1
        for j in range(_J):
            idxb[slot, j // _K, pl.ds((j % _K) * 16, 16)] = idx_list[j]
        for h in range(2):
            pltpu.async_copy(
                table_hbm.at[idxb.at[slot, h]],
                buf.at[pl.ds(slot * 2 * _HB + h * _HB, _HB)], sem)

    def drain(slot):
        sem = gs0 if slot == 0 else gs1
        for h in range(2):
            pltpu.make_async_copy(
                table_hbm.at[idxb.at[slot, h]],
                buf.at[pl.ds(slot * 2 * _HB + h * _HB, _HB)], sem).wait()

    idx0, w0 = calc(0)
    fire(idx0, 0)

    def body(s, wcur):
        p = s % 2
        @pl.when(jnp.logical_and(s >= 2, p == 0))
        def _():
            pltpu.make_async_copy(
                outbuf.at[pl.ds(0, 16)], out_hbm.at[pl.ds(0, 16)], os0).wait()

        @pl.when(jnp.logical_and(s >= 2, p == 1))
        def _():
            pltpu.make_async_copy(
                outbuf.at[pl.ds(16, 16)], out_hbm.at[pl.ds(0, 16)], os1).wait()

        s1 = jnp.minimum(s + 1, _NSTEP - 1)
        idxn, wnext = calc(s1)
        not_last = s < _NSTEP - 1

        @pl.when(jnp.logical_and(not_last, p == 0))
        def _():
            fire(idxn, 1)

        @pl.when(jnp.logical_and(not_last, p == 1))
        def _():
            fire(idxn, 0)

        @pl.when(p == 0)
        def _():
            drain(0)

        @pl.when(p == 1)
        def _():
            drain(1)

        rinv = wcur[_J]
        rowbase = p * (2 * _HB)

        def rbody(r, c):
            rsplat = jnp.full((16,), r, jnp.int32)
            wb = [jnp.take_along_axis(wcur[j], rsplat, axis=0)
                  for j in range(_J)]
            rb = jnp.take_along_axis(rinv, rsplat, axis=0)
            orow = p * 16 + r
            accs = [jnp.zeros((16,), jnp.float32) for _ in range(16)]
            for j in range(_J):
                row = rowbase + j * 16 + r
                for dblk in range(_GC2 // 16):
                    wrd = buf[row, pl.ds(dblk * 16, 16)]
                    accs[2 * dblk] = accs[2 * dblk] + wb[j] * plsc.bitcast(
                        lax.shift_left(wrd, 16), jnp.float32)
                    accs[2 * dblk + 1] = (
                        accs[2 * dblk + 1] + wb[j] * plsc.bitcast(
                            lax.bitwise_and(wrd, jnp.int32(-65536)),
                            jnp.float32))
            for dblk in range(_GC2 // 16):
                outbuf[orow, pl.ds(dblk * 32, 16)] = accs[2 * dblk] * rb
                outbuf[orow, pl.ds(dblk * 32 + 16, 16)] = (
                    accs[2 * dblk + 1] * rb)
            return c

        lax.fori_loop(0, 16, rbody, 0)

        t = s // _G
        g = s % _G
        orow = (n * _G + g) * _L + l0 + t * 16

        @pl.when(p == 0)
        def _():
            pltpu.async_copy(
                outbuf.at[pl.ds(0, 16)], out_hbm.at[pl.ds(orow, 16)], os0)

        @pl.when(p == 1)
        def _():
            pltpu.async_copy(
                outbuf.at[pl.ds(16, 16)], out_hbm.at[pl.ds(orow, 16)], os1)

        return wnext

    lax.fori_loop(0, _NSTEP, body, w0)
    pltpu.make_async_copy(
        outbuf.at[pl.ds(0, 16)], out_hbm.at[pl.ds(0, 16)], os0).wait()
    pltpu.make_async_copy(
        outbuf.at[pl.ds(16, 16)], out_hbm.at[pl.ds(0, 16)], os1).wait()


def _sc_gather(table_i32, om):
    mesh = plsc.VectorSubcoreMesh(core_axis_name="c", subcore_axis_name="s")
    f = functools.partial(
        pl.kernel,
        out_type=jax.ShapeDtypeStruct((_N * _G * _L, _GC), jnp.float32),
        mesh=mesh,
        compiler_params=pltpu.CompilerParams(
            use_tc_tiling_on_sc=False, needs_layout_passes=False),
        scratch_types=[
            pltpu.VMEM((2 * _G * _K, _LW), jnp.float32),
            pltpu.VMEM((2, 2, _HB), jnp.int32),
            pltpu.VMEM((4 * _HB, _GC2), jnp.int32),
            pltpu.VMEM((32, _GC), jnp.float32),
            pltpu.SemaphoreType.DMA,
            pltpu.SemaphoreType.DMA,
            pltpu.SemaphoreType.DMA,
            pltpu.SemaphoreType.DMA,
        ],
    )(_sc_body)
    return f(table_i32, om)


def _outproj_body(y, w, b, o_ref):
    g = pl.program_id(2)
    part = lax.dot_general(y[0, 0], w[0], (((1,), (0,)), ((), ())),
                           preferred_element_type=jnp.float32)

    @pl.when(g == 0)
    def _():
        o_ref[0] = part + b[...]

    @pl.when(g > 0)
    def _():
        o_ref[0] += part


def kernel(x, dw_w, dw_b, ln_g, ln_b, off_w, off_b, mask_w, mask_b,
           in_w, in_b, out_w, out_b):
    n, l, c = x.shape
    dww = jnp.transpose(dw_w[:, 0, :])
    omw = jnp.concatenate([off_w, mask_w], axis=0)
    omb = jnp.concatenate([off_b, mask_b], axis=0)[:, None]

    proj, om = pl.pallas_call(
        _prep_body,
        grid=(_N, _NIP),
        in_specs=[
            pl.BlockSpec((1, _LBP, _C), lambda n_, i: (n_, jnp.maximum(i - 1, 0), 0)),
            pl.BlockSpec((1, _LBP, _C), lambda n_, i: (n_, i, 0)),
            pl.BlockSpec((1, _LBP, _C), lambda n_, i: (n_, jnp.minimum(i + 1, _NIP - 1), 0)),
            pl.BlockSpec((3, _C), lambda n_, i: (0, 0)),
            pl.BlockSpec((1, _C), lambda n_, i: (0, 0)),
            pl.BlockSpec((1, _C), lambda n_, i: (0, 0)),
            pl.BlockSpec((1, _C), lambda n_, i: (0, 0)),
            pl.BlockSpec((_C, _C), lambda n_, i: (0, 0)),
            pl.BlockSpec((1, _C), lambda n_, i: (0, 0)),
            pl.BlockSpec((_G * _K * 2, _C), lambda n_, i: (0, 0)),
            pl.BlockSpec((_G * _K * 2, 1), lambda n_, i: (0, 0)),
        ],
        out_specs=[
            pl.BlockSpec((1, _G, _LBP, _GC), lambda n_, i: (n_, 0, i, 0)),
            pl.BlockSpec((1, _G * _K * 2, _LBP), lambda n_, i: (n_, 0, i)),
        ],
        out_shape=[
            jax.ShapeDtypeStruct((_N, _G, _L, _GC), jnp.bfloat16),
            jax.ShapeDtypeStruct((_N, _G * _K * 2, _L), jnp.float32),
        ],
    )(x, x, x, dww, dw_b[None], ln_g[None], ln_b[None], in_w, in_b[None],
      omw, omb)

    table_i32 = lax.bitcast_convert_type(
        proj.reshape(_N * _G * _L, _GC2, 2), jnp.int32)
    y = _sc_gather(table_i32, om)
    y = y.reshape(_N, _G, _L, _GC)

    ch = np.arange(_GC)
    i32b = ch % 32
    chanperm = (ch // 32) * 32 + np.where(i32b < 16, 2 * i32b,
                                          2 * (i32b - 16) + 1)
    w_r = jnp.transpose(out_w).reshape(_G, _GC, _C)[:, chanperm, :]
    o = pl.pallas_call(
        _outproj_body,
        grid=(_N, _NI, _G),
        in_specs=[
            pl.BlockSpec((1, 1, _LB, _GC), lambda n_, i, g: (n_, g, i, 0)),
            pl.BlockSpec((1, _GC, _C), lambda n_, i, g: (g, 0, 0)),
            pl.BlockSpec((1, _C), lambda n_, i, g: (0, 0)),
        ],
        out_specs=pl.BlockSpec((1, _LB, _C), lambda n_, i, g: (n_, i, 0)),
        out_shape=jax.ShapeDtypeStruct((_N, _L, _C), jnp.float32),
    )(y, w_r, out_b[None])
    return o


# speedup vs baseline: 5.0486x; 1.0045x over previous
import functools

import jax
import jax.numpy as jnp
from jax import lax
from jax.experimental import pallas as pl
from jax.experimental.pallas import tpu as pltpu
from jax.experimental.pallas import tpu_sc as plsc

_N, _L, _C = 2, 4096, 1024
_K, _G = 7, 4
_GC = _C // _G
_GC2 = _GC // 2
_SCALE = 2.0
_LB = 1024
_NI = _L // _LB
_LBP = 512
_NIP = _L // _LBP
_SQRT_HALF = 0.7071067811865476

_NC, _NS = 2, 16
_NW = _NC * _NS
_LW = (_N * _L) // _NW
_NT = _LW // 16
_NSTEP = _NT * _G
_J = 2 * _K


def _prep_body(xp, xc, xn, dww, dwb, lng, lnb, inw, inb, omw, omb,
               proj_ref, om_ref):
    i = pl.program_id(1)
    x = xc[0]
    left = jnp.where(i > 0, xp[0, _LBP - 1:_LBP, :], 0.0)
    right = jnp.where(i < _NIP - 1, xn[0, 0:1, :], 0.0)
    xm1 = jnp.concatenate([left, x[:-1]], axis=0)
    xp1 = jnp.concatenate([x[1:], right], axis=0)
    xdw = xm1 * dww[0:1] + x * dww[1:2] + xp1 * dww[2:3] + dwb[...]
    mu = jnp.mean(xdw, axis=-1, keepdims=True)
    xz = xdw - mu
    var = jnp.mean(xz * xz, axis=-1, keepdims=True)
    xdw = xz * lax.rsqrt(var + 1e-5) * lng[...] + lnb[...]
    xdw = 0.5 * xdw * (1.0 + lax.erf(xdw * _SQRT_HALF))
    proj = (lax.dot_general(x, inw[...], (((1,), (1,)), ((), ())),
                            preferred_element_type=jnp.float32) + inb[...])
    proj_ref[0] = jnp.transpose(proj.reshape(_LBP, _G, _GC), (1, 0, 2)).astype(
        jnp.bfloat16)
    om_ref[0] = (
        lax.dot_general(omw[...], xdw, (((1,), (1,)), ((), ())),
                        preferred_element_type=jnp.float32) + omb[...])


_HB = _J * 16 // 2


def _sc_body(table_hbm, om_hbm, out_hbm, om_v, idxb, buf, outbuf,
             gs0, gs1, os0, os1):
    wid = lax.axis_index("s") * _NC + lax.axis_index("c")
    n = wid // _NS
    l0 = (wid % _NS) * _LW
    pltpu.sync_copy(om_hbm.at[n, :, pl.ds(l0, _LW)], om_v)

    rowiota = lax.iota(jnp.int32, 16)
    fiota = rowiota.astype(jnp.float32)

    def calc(s):
        t = s // _G
        g = s % _G
        lbase = l0 + t * 16
        lvec = lax.convert_element_type(lbase, jnp.float32) + fiota
        base_row = (n * _G + g) * _L
        mraw = [om_v[_G * _K + g * _K + k, pl.ds(t * 16, 16)]
                for k in range(_K)]
        gmax = mraw[0]
        for k in range(1, _K):
            gmax = jnp.maximum(gmax, mraw[k])
        mexp = [jnp.exp(mr - gmax) for mr in mraw]
        msum = mexp[0]
        for k in range(1, _K):
            msum = msum + mexp[k]
        rinv = 1.0 / msum
        idx_list, w_list = [], []
        for k in range(_K):
            off = om_v[g * _K + k, pl.ds(t * 16, 16)] * _SCALE
            abs_pos = lvec + (k - _K // 2) + off
            apc = jnp.clip(abs_pos, 0.0, float(_L - 1))
            pf = apc.astype(jnp.int32)
            pc = jnp.minimum(pf + 1, _L - 1)
            wc = apc - pf.astype(jnp.float32)
            wf = 1.0 - wc
            valid = jnp.logical_and(abs_pos >= 0.0, abs_pos <= float(_L - 1))
            vm = jnp.where(valid, mexp[k], 0.0)
            w_list.append(wf * vm)
            w_list.append(wc * vm)
            idx_list.append(base_row + pf)
            idx_list.append(base_row + pc)
        return idx_list, tuple(w_list) + (rinv,)

    def fire(idx_list, slot):
        sem = gs0 if slot == 0 else gs1
        for j in range(_J):
            idxb[slot, j // _K, pl.ds((j % _K) * 16, 16)] = idx_list[j]
        for h in range(2):
            pltpu.async_copy(
                table_hbm.at[idxb.at[slot, h]],
                buf.at[pl.ds(slot * 2 * _HB + h * _HB, _HB)], sem)

    def drain(slot):
        sem = gs0 if slot == 0 else gs1
        for h in range(2):
            pltpu.make_async_copy(
                table_hbm.at[idxb.at[slot, h]],
                buf.at[pl.ds(slot * 2 * _HB + h * _HB, _HB)], sem).wait()

    idx0, w0 = calc(0)
    fire(idx0, 0)

    def body(s, wcur):
        p = s % 2
        @pl.when(jnp.logical_and(s >= 2, p == 0))
        def _():
            pltpu.make_async_copy(
                outbuf.at[pl.ds(0, 16)], out_hbm.at[pl.ds(0, 16)], os0).wait()

        @pl.when(jnp.logical_and(s >= 2, p == 1))
        def _():
            pltpu.make_async_copy(
                outbuf.at[pl.ds(16, 16)], out_hbm.at[pl.ds(0, 16)], os1).wait()

        s1 = jnp.minimum(s + 1, _NSTEP - 1)
        idxn, wnext = calc(s1)
        not_last = s < _NSTEP - 1

        @pl.when(jnp.logical_and(not_last, p == 0))
        def _():
            fire(idxn, 1)

        @pl.when(jnp.logical_and(not_last, p == 1))
        def _():
            fire(idxn, 0)

        @pl.when(p == 0)
        def _():
            drain(0)

        @pl.when(p == 1)
        def _():
            drain(1)

        rinv = wcur[_J]
        rowbase = p * (2 * _HB)

        def rbody(r, c):
            rsplat = jnp.full((16,), r, jnp.int32)
            wb = [jnp.take_along_axis(wcur[j], rsplat, axis=0)
                  for j in range(_J)]
            rb = jnp.take_along_axis(rinv, rsplat, axis=0)
            orow = p * 16 + r
            accs = [jnp.zeros((16,), jnp.float32) for _ in range(16)]
            for j in range(_J):
                row = rowbase + j * 16 + r
                for dblk in range(_GC2 // 16):
                    wrd = buf[row, pl.ds(dblk * 16, 16)]
                    accs[2 * dblk] = accs[2 * dblk] + wb[j] * plsc.bitcast(
                        lax.shift_left(wrd, 16), jnp.float32)
                    accs[2 * dblk + 1] = (
                        accs[2 * dblk + 1] + wb[j] * plsc.bitcast(
                            lax.bitwise_and(wrd, jnp.int32(-65536)),
                            jnp.float32))
            for dblk in range(_GC2 // 16):
                outbuf[orow, pl.ds(dblk * 32, 32)] = plsc.pack(
                    accs[2 * dblk] * rb, accs[2 * dblk + 1] * rb,
                    format=plsc.PackFormat.INTERLEAVED)
            return c

        lax.fori_loop(0, 16, rbody, 0)

        t = s // _G
        g = s % _G
        orow = (n * _G + g) * _L + l0 + t * 16

        @pl.when(p == 0)
        def _():
            pltpu.async_copy(
                outbuf.at[pl.ds(0, 16)], out_hbm.at[pl.ds(orow, 16)], os0)

        @pl.when(p == 1)
        def _():
            pltpu.async_copy(
                outbuf.at[pl.ds(16, 16)], out_hbm.at[pl.ds(orow, 16)], os1)

        return wnext

    lax.fori_loop(0, _NSTEP, body, w0)
    pltpu.make_async_copy(
        outbuf.at[pl.ds(0, 16)], out_hbm.at[pl.ds(0, 16)], os0).wait()
    pltpu.make_async_copy(
        outbuf.at[pl.ds(16, 16)], out_hbm.at[pl.ds(0, 16)], os1).wait()


def _sc_gather(table_i32, om):
    mesh = plsc.VectorSubcoreMesh(core_axis_name="c", subcore_axis_name="s")
    f = functools.partial(
        pl.kernel,
        out_type=jax.ShapeDtypeStruct((_N * _G * _L, _GC), jnp.bfloat16),
        mesh=mesh,
        compiler_params=pltpu.CompilerParams(
            use_tc_tiling_on_sc=False, needs_layout_passes=False),
        scratch_types=[
            pltpu.VMEM((2 * _G * _K, _LW), jnp.float32),
            pltpu.VMEM((2, 2, _HB), jnp.int32),
            pltpu.VMEM((4 * _HB, _GC2), jnp.int32),
            pltpu.VMEM((32, _GC), jnp.bfloat16),
            pltpu.SemaphoreType.DMA,
            pltpu.SemaphoreType.DMA,
            pltpu.SemaphoreType.DMA,
            pltpu.SemaphoreType.DMA,
        ],
    )(_sc_body)
    return f(table_i32, om)


def _outproj_body(y, w, b, o_ref):
    g = pl.program_id(2)
    part = lax.dot_general(y[0, 0].astype(jnp.float32), w[0],
                           (((1,), (0,)), ((), ())),
                           preferred_element_type=jnp.float32)

    @pl.when(g == 0)
    def _():
        o_ref[0] = part + b[...]

    @pl.when(g > 0)
    def _():
        o_ref[0] += part


def kernel(x, dw_w, dw_b, ln_g, ln_b, off_w, off_b, mask_w, mask_b,
           in_w, in_b, out_w, out_b):
    n, l, c = x.shape
    dww = jnp.transpose(dw_w[:, 0, :])
    omw = jnp.concatenate([off_w, mask_w], axis=0)
    omb = jnp.concatenate([off_b, mask_b], axis=0)[:, None]

    proj, om = pl.pallas_call(
        _prep_body,
        grid=(_N, _NIP),
        in_specs=[
            pl.BlockSpec((1, _LBP, _C), lambda n_, i: (n_, jnp.maximum(i - 1, 0), 0)),
            pl.BlockSpec((1, _LBP, _C), lambda n_, i: (n_, i, 0)),
            pl.BlockSpec((1, _LBP, _C), lambda n_, i: (n_, jnp.minimum(i + 1, _NIP - 1), 0)),
            pl.BlockSpec((3, _C), lambda n_, i: (0, 0)),
            pl.BlockSpec((1, _C), lambda n_, i: (0, 0)),
            pl.BlockSpec((1, _C), lambda n_, i: (0, 0)),
            pl.BlockSpec((1, _C), lambda n_, i: (0, 0)),
            pl.BlockSpec((_C, _C), lambda n_, i: (0, 0)),
            pl.BlockSpec((1, _C), lambda n_, i: (0, 0)),
            pl.BlockSpec((_G * _K * 2, _C), lambda n_, i: (0, 0)),
            pl.BlockSpec((_G * _K * 2, 1), lambda n_, i: (0, 0)),
        ],
        out_specs=[
            pl.BlockSpec((1, _G, _LBP, _GC), lambda n_, i: (n_, 0, i, 0)),
            pl.BlockSpec((1, _G * _K * 2, _LBP), lambda n_, i: (n_, 0, i)),
        ],
        out_shape=[
            jax.ShapeDtypeStruct((_N, _G, _L, _GC), jnp.bfloat16),
            jax.ShapeDtypeStruct((_N, _G * _K * 2, _L), jnp.float32),
        ],
    )(x, x, x, dww, dw_b[None], ln_g[None], ln_b[None], in_w, in_b[None],
      omw, omb)

    table_i32 = lax.bitcast_convert_type(
        proj.reshape(_N * _G * _L, _GC2, 2), jnp.int32)
    y = _sc_gather(table_i32, om)
    y = y.reshape(_N, _G, _L, _GC)

    w_r = jnp.transpose(out_w).reshape(_G, _GC, _C)
    o = pl.pallas_call(
        _outproj_body,
        grid=(_N, _NI, _G),
        in_specs=[
            pl.BlockSpec((1, 1, _LB, _GC), lambda n_, i, g: (n_, g, i, 0)),
            pl.BlockSpec((1, _GC, _C), lambda n_, i, g: (g, 0, 0)),
            pl.BlockSpec((1, _C), lambda n_, i, g: (0, 0)),
        ],
        out_specs=pl.BlockSpec((1, _LB, _C), lambda n_, i, g: (n_, i, 0)),
        out_shape=jax.ShapeDtypeStruct((_N, _L, _C), jnp.float32),
    )(y, w_r, out_b[None])
    return o


# speedup vs baseline: 6.6672x; 1.3206x over previous
import functools

import jax
import jax.numpy as jnp
from jax import lax
from jax.experimental import pallas as pl
from jax.experimental.pallas import tpu as pltpu
from jax.experimental.pallas import tpu_sc as plsc

_N, _L, _C = 2, 4096, 1024
_K, _G = 7, 4
_GC = _C // _G
_GC2 = _GC // 2
_SCALE = 2.0
_LB = 1024
_NI = _L // _LB
_LBP = 512
_NIP = _L // _LBP
_SQRT_HALF = 0.7071067811865476

_NC, _NS = 2, 16
_NW = _NC * _NS
_LW = (_N * _L) // _NW
_NT = _LW // 16
_NSTEP = _NT * _G
_J = 2 * _K


def _prep_body(xp, xc, xn, dww, dwb, lng, lnb, inw, inb, omw, omb,
               proj_ref, om_ref):
    i = pl.program_id(1)
    x = xc[0]
    left = jnp.where(i > 0, xp[0, _LBP - 1:_LBP, :], 0.0)
    right = jnp.where(i < _NIP - 1, xn[0, 0:1, :], 0.0)
    xm1 = jnp.concatenate([left, x[:-1]], axis=0)
    xp1 = jnp.concatenate([x[1:], right], axis=0)
    xdw = xm1 * dww[0:1] + x * dww[1:2] + xp1 * dww[2:3] + dwb[...]
    mu = jnp.mean(xdw, axis=-1, keepdims=True)
    xz = xdw - mu
    var = jnp.mean(xz * xz, axis=-1, keepdims=True)
    xdw = xz * lax.rsqrt(var + 1e-5) * lng[...] + lnb[...]
    xdw = 0.5 * xdw * (1.0 + lax.erf(xdw * _SQRT_HALF))
    proj = (lax.dot_general(x, inw[...], (((1,), (1,)), ((), ())),
                            preferred_element_type=jnp.float32) + inb[...])
    proj_ref[0] = jnp.transpose(proj.reshape(_LBP, _G, _GC), (1, 0, 2)).astype(
        jnp.bfloat16)
    om_ref[0] = (
        lax.dot_general(omw[...], xdw, (((1,), (1,)), ((), ())),
                        preferred_element_type=jnp.float32) + omb[...])


_HB = _J * 16 // 2


def _sc_body(table_hbm, om_hbm, out_hbm, om_v, idxb, buf, outbuf,
             gs0, gs1, os0, os1):
    wid = lax.axis_index("s") * _NC + lax.axis_index("c")
    n = wid // _NS
    l0 = (wid % _NS) * _LW
    pltpu.sync_copy(om_hbm.at[n, :, pl.ds(l0, _LW)], om_v)

    rowiota = lax.iota(jnp.int32, 16)
    fiota = rowiota.astype(jnp.float32)

    def calc(s):
        t = s // _G
        g = s % _G
        lbase = l0 + t * 16
        lvec = lax.convert_element_type(lbase, jnp.float32) + fiota
        base_row = (n * _G + g) * _L
        mraw = [om_v[_G * _K + g * _K + k, pl.ds(t * 16, 16)]
                for k in range(_K)]
        gmax = mraw[0]
        for k in range(1, _K):
            gmax = jnp.maximum(gmax, mraw[k])
        mexp = [jnp.exp(mr - gmax) for mr in mraw]
        msum = mexp[0]
        for k in range(1, _K):
            msum = msum + mexp[k]
        rinv = 1.0 / msum
        idx_list, w_list = [], []
        for k in range(_K):
            off = om_v[g * _K + k, pl.ds(t * 16, 16)] * _SCALE
            abs_pos = lvec + (k - _K // 2) + off
            apc = jnp.clip(abs_pos, 0.0, float(_L - 1))
            pf = apc.astype(jnp.int32)
            pc = jnp.minimum(pf + 1, _L - 1)
            wc = apc - pf.astype(jnp.float32)
            wf = 1.0 - wc
            valid = jnp.logical_and(abs_pos >= 0.0, abs_pos <= float(_L - 1))
            vm = jnp.where(valid, mexp[k], 0.0)
            w_list.append(wf * vm)
            w_list.append(wc * vm)
            idx_list.append(base_row + pf)
            idx_list.append(base_row + pc)
        return idx_list, tuple(w_list) + (rinv,)

    def fire(idx_list, slot):
        sem = gs0 if slot == 0 else gs1
        for j in range(_J):
            idxb[slot, j // _K, pl.ds((j % _K) * 16, 16)] = idx_list[j]
        for h in range(2):
            pltpu.async_copy(
                table_hbm.at[idxb.at[slot, h]],
                buf.at[pl.ds(slot * 2 * _HB + h * _HB, _HB)], sem)

    def drain(slot):
        sem = gs0 if slot == 0 else gs1
        for h in range(2):
            pltpu.make_async_copy(
                table_hbm.at[idxb.at[slot, h]],
                buf.at[pl.ds(slot * 2 * _HB + h * _HB, _HB)], sem).wait()

    idx0, w0 = calc(0)
    fire(idx0, 0)

    def body(s, wcur):
        p = s % 2
        @pl.when(jnp.logical_and(s >= 2, p == 0))
        def _():
            pltpu.make_async_copy(
                outbuf.at[pl.ds(0, 16)], out_hbm.at[pl.ds(0, 16)], os0).wait()

        @pl.when(jnp.logical_and(s >= 2, p == 1))
        def _():
            pltpu.make_async_copy(
                outbuf.at[pl.ds(16, 16)], out_hbm.at[pl.ds(0, 16)], os1).wait()

        s1 = jnp.minimum(s + 1, _NSTEP - 1)
        idxn, wnext = calc(s1)
        not_last = s < _NSTEP - 1

        @pl.when(jnp.logical_and(not_last, p == 0))
        def _():
            fire(idxn, 1)

        @pl.when(jnp.logical_and(not_last, p == 1))
        def _():
            fire(idxn, 0)

        @pl.when(p == 0)
        def _():
            drain(0)

        @pl.when(p == 1)
        def _():
            drain(1)

        rinv = wcur[_J]
        rowbase = p * (2 * _HB)

        def rbody(r, c):
            rsplat = jnp.full((16,), r, jnp.int32)
            wb = [jnp.take_along_axis(wcur[j], rsplat, axis=0)
                  for j in range(_J)]
            rb = jnp.take_along_axis(rinv, rsplat, axis=0)
            orow = p * 16 + r
            accs = [jnp.zeros((16,), jnp.float32) for _ in range(16)]
            for j in range(_J):
                row = rowbase + j * 16 + r
                for dblk in range(_GC2 // 16):
                    pair = buf[row, pl.ds(dblk * 32, 32)]
                    ev, od = plsc.unpack(pair, format=plsc.PackFormat.INTERLEAVED)
                    accs[2 * dblk] = accs[2 * dblk] + wb[j] * ev
                    accs[2 * dblk + 1] = accs[2 * dblk + 1] + wb[j] * od
            for dblk in range(_GC2 // 16):
                outbuf[orow, pl.ds(dblk * 32, 32)] = plsc.pack(
                    accs[2 * dblk] * rb, accs[2 * dblk + 1] * rb,
                    format=plsc.PackFormat.INTERLEAVED)
            return c

        lax.fori_loop(0, 16, rbody, 0)

        t = s // _G
        g = s % _G
        orow = (n * _G + g) * _L + l0 + t * 16

        @pl.when(p == 0)
        def _():
            pltpu.async_copy(
                outbuf.at[pl.ds(0, 16)], out_hbm.at[pl.ds(orow, 16)], os0)

        @pl.when(p == 1)
        def _():
            pltpu.async_copy(
                outbuf.at[pl.ds(16, 16)], out_hbm.at[pl.ds(orow, 16)], os1)

        return wnext

    lax.fori_loop(0, _NSTEP, body, w0)
    pltpu.make_async_copy(
        outbuf.at[pl.ds(0, 16)], out_hbm.at[pl.ds(0, 16)], os0).wait()
    pltpu.make_async_copy(
        outbuf.at[pl.ds(16, 16)], out_hbm.at[pl.ds(0, 16)], os1).wait()


def _sc_gather(table_i32, om):
    mesh = plsc.VectorSubcoreMesh(core_axis_name="c", subcore_axis_name="s")
    f = functools.partial(
        pl.kernel,
        out_type=jax.ShapeDtypeStruct((_N * _G * _L, _GC), jnp.bfloat16),
        mesh=mesh,
        compiler_params=pltpu.CompilerParams(
            use_tc_tiling_on_sc=False, needs_layout_passes=False),
        scratch_types=[
            pltpu.VMEM((2 * _G * _K, _LW), jnp.float32),
            pltpu.VMEM((2, 2, _HB), jnp.int32),
            pltpu.VMEM((4 * _HB, _GC), jnp.bfloat16),
            pltpu.VMEM((32, _GC), jnp.bfloat16),
            pltpu.SemaphoreType.DMA,
            pltpu.SemaphoreType.DMA,
            pltpu.SemaphoreType.DMA,
            pltpu.SemaphoreType.DMA,
        ],
    )(_sc_body)
    return f(table_i32, om)


def _outproj_body(y, w, b, o_ref):
    g = pl.program_id(2)
    part = lax.dot_general(y[0, 0].astype(jnp.float32), w[0],
                           (((1,), (0,)), ((), ())),
                           preferred_element_type=jnp.float32)

    @pl.when(g == 0)
    def _():
        o_ref[0] = part + b[...]

    @pl.when(g > 0)
    def _():
        o_ref[0] += part


def kernel(x, dw_w, dw_b, ln_g, ln_b, off_w, off_b, mask_w, mask_b,
           in_w, in_b, out_w, out_b):
    n, l, c = x.shape
    dww = jnp.transpose(dw_w[:, 0, :])
    omw = jnp.concatenate([off_w, mask_w], axis=0)
    omb = jnp.concatenate([off_b, mask_b], axis=0)[:, None]

    proj, om = pl.pallas_call(
        _prep_body,
        grid=(_N, _NIP),
        in_specs=[
            pl.BlockSpec((1, _LBP, _C), lambda n_, i: (n_, jnp.maximum(i - 1, 0), 0)),
            pl.BlockSpec((1, _LBP, _C), lambda n_, i: (n_, i, 0)),
            pl.BlockSpec((1, _LBP, _C), lambda n_, i: (n_, jnp.minimum(i + 1, _NIP - 1), 0)),
            pl.BlockSpec((3, _C), lambda n_, i: (0, 0)),
            pl.BlockSpec((1, _C), lambda n_, i: (0, 0)),
            pl.BlockSpec((1, _C), lambda n_, i: (0, 0)),
            pl.BlockSpec((1, _C), lambda n_, i: (0, 0)),
            pl.BlockSpec((_C, _C), lambda n_, i: (0, 0)),
            pl.BlockSpec((1, _C), lambda n_, i: (0, 0)),
            pl.BlockSpec((_G * _K * 2, _C), lambda n_, i: (0, 0)),
            pl.BlockSpec((_G * _K * 2, 1), lambda n_, i: (0, 0)),
        ],
        out_specs=[
            pl.BlockSpec((1, _G, _LBP, _GC), lambda n_, i: (n_, 0, i, 0)),
            pl.BlockSpec((1, _G * _K * 2, _LBP), lambda n_, i: (n_, 0, i)),
        ],
        out_shape=[
            jax.ShapeDtypeStruct((_N, _G, _L, _GC), jnp.bfloat16),
            jax.ShapeDtypeStruct((_N, _G * _K * 2, _L), jnp.float32),
        ],
    )(x, x, x, dww, dw_b[None], ln_g[None], ln_b[None], in_w, in_b[None],
      omw, omb)

    y = _sc_gather(proj.reshape(_N * _G * _L, _GC), om)
    y = y.reshape(_N, _G, _L, _GC)

    w_r = jnp.transpose(out_w).reshape(_G, _GC, _C)
    o = pl.pallas_call(
        _outproj_body,
        grid=(_N, _NI, _G),
        in_specs=[
            pl.BlockSpec((1, 1, _LB, _GC), lambda n_, i, g: (n_, g, i, 0)),
            pl.BlockSpec((1, _GC, _C), lambda n_, i, g: (g, 0, 0)),
            pl.BlockSpec((1, _C), lambda n_, i, g: (0, 0)),
        ],
        out_specs=pl.BlockSpec((1, _LB, _C), lambda n_, i, g: (n_, i, 0)),
        out_shape=jax.ShapeDtypeStruct((_N, _L, _C), jnp.float32),
    )(y, w_r, out_b[None])
    return o


# speedup vs baseline: 6.6673x; 1.0000x over previous
import functools

import jax
import jax.numpy as jnp
from jax import lax
from jax.experimental import pallas as pl
from jax.experimental.pallas import tpu as pltpu
from jax.experimental.pallas import tpu_sc as plsc

_N, _L, _C = 2, 4096, 1024
_K, _G = 7, 4
_GC = _C // _G
_GC2 = _GC // 2
_SCALE = 2.0
_LB = 1024
_NI = _L // _LB
_LBP = 512
_NIP = _L // _LBP
_SQRT_HALF = 0.7071067811865476

_NC, _NS = 2, 16
_NW = _NC * _NS
_LW = (_N * _L) // _NW
_NT = _LW // 16
_NSTEP = _NT * _G
_J = 2 * _K


def _prep_body(xp, xc, xn, dww, dwb, lng, lnb, inw, inb, omw, omb,
               proj_ref, om_ref):
    i = pl.program_id(1)
    x = xc[0]
    left = jnp.where(i > 0, xp[0, _LBP - 1:_LBP, :], 0.0)
    right = jnp.where(i < _NIP - 1, xn[0, 0:1, :], 0.0)
    xm1 = jnp.concatenate([left, x[:-1]], axis=0)
    xp1 = jnp.concatenate([x[1:], right], axis=0)
    xdw = xm1 * dww[0:1] + x * dww[1:2] + xp1 * dww[2:3] + dwb[...]
    mu = jnp.mean(xdw, axis=-1, keepdims=True)
    xz = xdw - mu
    var = jnp.mean(xz * xz, axis=-1, keepdims=True)
    xdw = xz * lax.rsqrt(var + 1e-5) * lng[...] + lnb[...]
    xdw = 0.5 * xdw * (1.0 + lax.erf(xdw * _SQRT_HALF))
    proj = (lax.dot_general(x, inw[...], (((1,), (1,)), ((), ())),
                            preferred_element_type=jnp.float32) + inb[...])
    proj_ref[0] = jnp.transpose(proj.reshape(_LBP, _G, _GC), (1, 0, 2)).astype(
        jnp.bfloat16)
    om_ref[0] = (
        lax.dot_general(omw[...], xdw, (((1,), (1,)), ((), ())),
                        preferred_element_type=jnp.float32) + omb[...])


_HB = _J * 16 // 2


def _sc_body(table_hbm, om_hbm, out_hbm, om_v, idxb, buf, outbuf,
             gs0, gs1, os0, os1):
    wid = lax.axis_index("s") * _NC + lax.axis_index("c")
    n = wid // _NS
    l0 = (wid % _NS) * _LW
    pltpu.sync_copy(om_hbm.at[n, :, pl.ds(l0, _LW)], om_v)

    rowiota = lax.iota(jnp.int32, 16)
    fiota = rowiota.astype(jnp.float32)

    def calc(s):
        t = s // _G
        g = s % _G
        lbase = l0 + t * 16
        lvec = lax.convert_element_type(lbase, jnp.float32) + fiota
        base_row = (n * _G + g) * _L
        mraw = [om_v[_G * _K + g * _K + k, pl.ds(t * 16, 16)]
                for k in range(_K)]
        gmax = mraw[0]
        for k in range(1, _K):
            gmax = jnp.maximum(gmax, mraw[k])
        mexp = [jnp.exp(mr - gmax) for mr in mraw]
        msum = mexp[0]
        for k in range(1, _K):
            msum = msum + mexp[k]
        rinv = 1.0 / msum
        idx_list, w_list = [], []
        for k in range(_K):
            off = om_v[g * _K + k, pl.ds(t * 16, 16)] * _SCALE
            abs_pos = lvec + (k - _K // 2) + off
            apc = jnp.clip(abs_pos, 0.0, float(_L - 1))
            pf = apc.astype(jnp.int32)
            pc = jnp.minimum(pf + 1, _L - 1)
            wc = apc - pf.astype(jnp.float32)
            wf = 1.0 - wc
            valid = jnp.logical_and(abs_pos >= 0.0, abs_pos <= float(_L - 1))
            vm = jnp.where(valid, mexp[k], 0.0)
            w_list.append(wf * vm)
            w_list.append(wc * vm)
            idx_list.append(base_row + pf)
            idx_list.append(base_row + pc)
        return idx_list, tuple(w_list) + (rinv,)

    def fire(idx_list, slot):
        sem = gs0 if slot == 0 else gs1
        for j in range(_J):
            idxb[slot, j // _K, pl.ds((j % _K) * 16, 16)] = idx_list[j]
        for h in range(2):
            pltpu.async_copy(
                table_hbm.at[idxb.at[slot, h]],
                buf.at[pl.ds(slot * 2 * _HB + h * _HB, _HB)], sem)

    def drain(slot):
        sem = gs0 if slot == 0 else gs1
        for h in range(2):
            pltpu.make_async_copy(
                table_hbm.at[idxb.at[slot, h]],
                buf.at[pl.ds(slot * 2 * _HB + h * _HB, _HB)], sem).wait()

    idx0, w0 = calc(0)
    fire(idx0, 0)

    def body(s, wcur):
        p = s % 2
        @pl.when(jnp.logical_and(s >= 2, p == 0))
        def _():
            pltpu.make_async_copy(
                outbuf.at[pl.ds(0, 16)], out_hbm.at[pl.ds(0, 16)], os0).wait()

        @pl.when(jnp.logical_and(s >= 2, p == 1))
        def _():
            pltpu.make_async_copy(
                outbuf.at[pl.ds(16, 16)], out_hbm.at[pl.ds(0, 16)], os1).wait()

        s1 = jnp.minimum(s + 1, _NSTEP - 1)
        idxn, wnext = calc(s1)
        not_last = s < _NSTEP - 1

        @pl.when(jnp.logical_and(not_last, p == 0))
        def _():
            fire(idxn, 1)

        @pl.when(jnp.logical_and(not_last, p == 1))
        def _():
            fire(idxn, 0)

        @pl.when(p == 0)
        def _():
            drain(0)

        @pl.when(p == 1)
        def _():
            drain(1)

        rinv = wcur[_J]
        rowbase = p * (2 * _HB)

        def rbody(r, c):
            rsplat = jnp.full((16,), r, jnp.int32)
            wb = [jnp.take_along_axis(wcur[j], rsplat, axis=0)
                  for j in range(_J)]
            rb = jnp.take_along_axis(rinv, rsplat, axis=0)
            orow = p * 16 + r
            accs = [jnp.zeros((16,), jnp.float32) for _ in range(16)]
            for j in range(_J):
                row = rowbase + j * 16 + r
                for dblk in range(_GC2 // 16):
                    pair = buf[row, pl.ds(dblk * 32, 32)]
                    ev, od = plsc.unpack(pair, format=plsc.PackFormat.INTERLEAVED)
                    accs[2 * dblk] = accs[2 * dblk] + wb[j] * ev
                    accs[2 * dblk + 1] = accs[2 * dblk + 1] + wb[j] * od
            for dblk in range(_GC2 // 16):
                outbuf[orow, pl.ds(dblk * 32, 32)] = plsc.pack(
                    accs[2 * dblk] * rb, accs[2 * dblk + 1] * rb,
                    format=plsc.PackFormat.INTERLEAVED)
            return c

        lax.fori_loop(0, 16, rbody, 0)

        t = s // _G
        g = s % _G
        orow = (n * _G + g) * _L + l0 + t * 16

        @pl.when(p == 0)
        def _():
            pltpu.async_copy(
                outbuf.at[pl.ds(0, 16)], out_hbm.at[pl.ds(orow, 16)], os0)

        @pl.when(p == 1)
        def _():
            pltpu.async_copy(
                outbuf.at[pl.ds(16, 16)], out_hbm.at[pl.ds(orow, 16)], os1)

        return wnext

    lax.fori_loop(0, _NSTEP, body, w0)
    pltpu.make_async_copy(
        outbuf.at[pl.ds(0, 16)], out_hbm.at[pl.ds(0, 16)], os0).wait()
    pltpu.make_async_copy(
        outbuf.at[pl.ds(16, 16)], out_hbm.at[pl.ds(0, 16)], os1).wait()


def _sc_gather(table, om):
    mesh = plsc.VectorSubcoreMesh(core_axis_name="c", subcore_axis_name="s")
    f = functools.partial(
        pl.kernel,
        out_type=jax.ShapeDtypeStruct((_N * _G * _L, _GC), jnp.bfloat16),
        mesh=mesh,
        compiler_params=pltpu.CompilerParams(
            use_tc_tiling_on_sc=False, needs_layout_passes=False),
        scratch_types=[
            pltpu.VMEM((2 * _G * _K, _LW), jnp.float32),
            pltpu.VMEM((2, 2, _HB), jnp.int32),
            pltpu.VMEM((4 * _HB, _GC), jnp.bfloat16),
            pltpu.VMEM((32, _GC), jnp.bfloat16),
            pltpu.SemaphoreType.DMA,
            pltpu.SemaphoreType.DMA,
            pltpu.SemaphoreType.DMA,
            pltpu.SemaphoreType.DMA,
        ],
    )(_sc_body)
    return f(table, om)


def _outproj_body(y, w, b, o_ref):
    g = pl.program_id(2)
    part = lax.dot_general(y[0, 0].astype(jnp.float32), w[0],
                           (((1,), (0,)), ((), ())),
                           preferred_element_type=jnp.float32)

    @pl.when(g == 0)
    def _():
        o_ref[0] = part + b[...]

    @pl.when(g > 0)
    def _():
        o_ref[0] += part


def kernel(x, dw_w, dw_b, ln_g, ln_b, off_w, off_b, mask_w, mask_b,
           in_w, in_b, out_w, out_b):
    n, l, c = x.shape
    dww = jnp.transpose(dw_w[:, 0, :])
    omw = jnp.concatenate([off_w, mask_w], axis=0)
    omb = jnp.concatenate([off_b, mask_b], axis=0)[:, None]

    proj, om = pl.pallas_call(
        _prep_body,
        grid=(_N, _NIP),
        in_specs=[
            pl.BlockSpec((1, _LBP, _C), lambda n_, i: (n_, jnp.maximum(i - 1, 0), 0)),
            pl.BlockSpec((1, _LBP, _C), lambda n_, i: (n_, i, 0)),
            pl.BlockSpec((1, _LBP, _C), lambda n_, i: (n_, jnp.minimum(i + 1, _NIP - 1), 0)),
            pl.BlockSpec((3, _C), lambda n_, i: (0, 0)),
            pl.BlockSpec((1, _C), lambda n_, i: (0, 0)),
            pl.BlockSpec((1, _C), lambda n_, i: (0, 0)),
            pl.BlockSpec((1, _C), lambda n_, i: (0, 0)),
            pl.BlockSpec((_C, _C), lambda n_, i: (0, 0)),
            pl.BlockSpec((1, _C), lambda n_, i: (0, 0)),
            pl.BlockSpec((_G * _K * 2, _C), lambda n_, i: (0, 0)),
            pl.BlockSpec((_G * _K * 2, 1), lambda n_, i: (0, 0)),
        ],
        out_specs=[
            pl.BlockSpec((1, _G, _LBP, _GC), lambda n_, i: (n_, 0, i, 0)),
            pl.BlockSpec((1, _G * _K * 2, _LBP), lambda n_, i: (n_, 0, i)),
        ],
        out_shape=[
            jax.ShapeDtypeStruct((_N, _G, _L, _GC), jnp.bfloat16),
            jax.ShapeDtypeStruct((_N, _G * _K * 2, _L), jnp.float32),
        ],
    )(x, x, x, dww, dw_b[None], ln_g[None], ln_b[None], in_w, in_b[None],
      omw, omb)

    y = _sc_gather(proj.reshape(_N * _G * _L, _GC), om)
    y = y.reshape(_N, _G, _L, _GC)

    w_r = jnp.transpose(out_w).reshape(_G, _GC, _C)
    o = pl.pallas_call(
        _outproj_body,
        grid=(_N, _NI, _G),
        in_specs=[
            pl.BlockSpec((1, 1, _LB, _GC), lambda n_, i, g: (n_, g, i, 0)),
            pl.BlockSpec((1, _GC, _C), lambda n_, i, g: (g, 0, 0)),
            pl.BlockSpec((1, _C), lambda n_, i, g: (0, 0)),
        ],
        out_specs=pl.BlockSpec((1, _LB, _C), lambda n_, i, g: (n_, i, 0)),
        out_shape=jax.ShapeDtypeStruct((_N, _L, _C), jnp.float32),
    )(y, w_r, out_b[None])
    return o


# speedup vs baseline: 7.0819x; 1.0622x over previous
import functools

import jax
import jax.numpy as jnp
from jax import lax
from jax.experimental import pallas as pl
from jax.experimental.pallas import tpu as pltpu
from jax.experimental.pallas import tpu_sc as plsc

_N, _L, _C = 2, 4096, 1024
_K, _G = 7, 4
_GC = _C // _G
_GC2 = _GC // 2
_SCALE = 2.0
_LB = 1024
_NI = _L // _LB
_LBP = 512
_NIP = _L // _LBP
_SQRT_HALF = 0.7071067811865476

_NC, _NS = 2, 16
_NW = _NC * _NS
_LW = (_N * _L) // _NW
_NT = _LW // 16
_NSTEP = _NT * _G
_J = 2 * _K


def _prep_body(xp, xc, xn, dww, dwb, lng, lnb, inw, inb, omw, omb,
               proj_ref, om_ref):
    i = pl.program_id(1)
    x = xc[0]
    left = jnp.where(i > 0, xp[0, _LBP - 1:_LBP, :], 0.0)
    right = jnp.where(i < _NIP - 1, xn[0, 0:1, :], 0.0)
    xm1 = jnp.concatenate([left, x[:-1]], axis=0)
    xp1 = jnp.concatenate([x[1:], right], axis=0)
    xdw = xm1 * dww[0:1] + x * dww[1:2] + xp1 * dww[2:3] + dwb[...]
    mu = jnp.mean(xdw, axis=-1, keepdims=True)
    xz = xdw - mu
    var = jnp.mean(xz * xz, axis=-1, keepdims=True)
    xdw = xz * lax.rsqrt(var + 1e-5) * lng[...] + lnb[...]
    xdw = 0.5 * xdw * (1.0 + lax.erf(xdw * _SQRT_HALF))
    proj = (lax.dot_general(x, inw[...], (((1,), (1,)), ((), ())),
                            preferred_element_type=jnp.float32) + inb[...])
    proj_ref[0] = jnp.transpose(proj.reshape(_LBP, _G, _GC), (1, 0, 2)).astype(
        jnp.bfloat16)
    om_ref[0] = (
        lax.dot_general(omw[...], xdw, (((1,), (1,)), ((), ())),
                        preferred_element_type=jnp.float32) + omb[...])


_HB = _J * 16 // 2


def _sc_body(table_hbm, om_hbm, out_hbm, om_v, idxb, buf, outbuf,
             gs0, gs1, gs2, os0, os1):
    wid = lax.axis_index("s") * _NC + lax.axis_index("c")
    n = wid // _NS
    l0 = (wid % _NS) * _LW
    pltpu.sync_copy(om_hbm.at[n, :, pl.ds(l0, _LW)], om_v)

    rowiota = lax.iota(jnp.int32, 16)
    fiota = rowiota.astype(jnp.float32)

    def calc(s):
        t = s // _G
        g = s % _G
        lbase = l0 + t * 16
        lvec = lax.convert_element_type(lbase, jnp.float32) + fiota
        base_row = (n * _G + g) * _L
        mraw = [om_v[_G * _K + g * _K + k, pl.ds(t * 16, 16)]
                for k in range(_K)]
        gmax = mraw[0]
        for k in range(1, _K):
            gmax = jnp.maximum(gmax, mraw[k])
        mexp = [jnp.exp(mr - gmax) for mr in mraw]
        msum = mexp[0]
        for k in range(1, _K):
            msum = msum + mexp[k]
        rinv = 1.0 / msum
        idx_list, w_list = [], []
        for k in range(_K):
            off = om_v[g * _K + k, pl.ds(t * 16, 16)] * _SCALE
            abs_pos = lvec + (k - _K // 2) + off
            apc = jnp.clip(abs_pos, 0.0, float(_L - 1))
            pf = apc.astype(jnp.int32)
            pc = jnp.minimum(pf + 1, _L - 1)
            wc = apc - pf.astype(jnp.float32)
            wf = 1.0 - wc
            valid = jnp.logical_and(abs_pos >= 0.0, abs_pos <= float(_L - 1))
            vm = jnp.where(valid, mexp[k], 0.0)
            w_list.append(wf * vm)
            w_list.append(wc * vm)
            idx_list.append(base_row + pf)
            idx_list.append(base_row + pc)
        return idx_list, tuple(w_list) + (rinv,)

    def fire(idx_list, slot, sem):
        for j in range(_J):
            idxb[slot, j // _K, pl.ds((j % _K) * 16, 16)] = idx_list[j]
        for h in range(2):
            pltpu.async_copy(
                table_hbm.at[idxb.at[slot, h]],
                buf.at[pl.ds(slot * 2 * _HB + h * _HB, _HB)], sem)

    def drain(slot, sem):
        for h in range(2):
            pltpu.make_async_copy(
                table_hbm.at[idxb.at[slot, h]],
                buf.at[pl.ds(slot * 2 * _HB + h * _HB, _HB)], sem).wait()

    idx0, _ = calc(0)
    fire(idx0, 0, gs0)
    idx1, _ = calc(1)
    fire(idx1, 1, gs1)

    def body(s, carry):
        p = s % 2
        m3 = s % 3
        @pl.when(jnp.logical_and(s >= 2, p == 0))
        def _():
            pltpu.make_async_copy(
                outbuf.at[pl.ds(0, 16)], out_hbm.at[pl.ds(0, 16)], os0).wait()

        @pl.when(jnp.logical_and(s >= 2, p == 1))
        def _():
            pltpu.make_async_copy(
                outbuf.at[pl.ds(16, 16)], out_hbm.at[pl.ds(0, 16)], os1).wait()

        s2 = jnp.minimum(s + 2, _NSTEP - 1)
        idxn, _ = calc(s2)
        ok2 = s < _NSTEP - 2

        @pl.when(jnp.logical_and(ok2, m3 == 0))
        def _():
            fire(idxn, 2, gs2)

        @pl.when(jnp.logical_and(ok2, m3 == 1))
        def _():
            fire(idxn, 0, gs0)

        @pl.when(jnp.logical_and(ok2, m3 == 2))
        def _():
            fire(idxn, 1, gs1)

        @pl.when(m3 == 0)
        def _():
            drain(0, gs0)

        @pl.when(m3 == 1)
        def _():
            drain(1, gs1)

        @pl.when(m3 == 2)
        def _():
            drain(2, gs2)

        _, wcur = calc(s)
        rinv = wcur[_J]
        rowbase = m3 * (2 * _HB)

        def rbody(r, c):
            rsplat = jnp.full((16,), r, jnp.int32)
            wb = [jnp.take_along_axis(wcur[j], rsplat, axis=0)
                  for j in range(_J)]
            rb = jnp.take_along_axis(rinv, rsplat, axis=0)
            orow = p * 16 + r
            accs = [jnp.zeros((16,), jnp.float32) for _ in range(16)]
            for j in range(_J):
                row = rowbase + j * 16 + r
                for dblk in range(_GC2 // 16):
                    pair = buf[row, pl.ds(dblk * 32, 32)]
                    ev, od = plsc.unpack(pair, format=plsc.PackFormat.INTERLEAVED)
                    accs[2 * dblk] = accs[2 * dblk] + wb[j] * ev
                    accs[2 * dblk + 1] = accs[2 * dblk + 1] + wb[j] * od
            for dblk in range(_GC2 // 16):
                outbuf[orow, pl.ds(dblk * 32, 32)] = plsc.pack(
                    accs[2 * dblk] * rb, accs[2 * dblk + 1] * rb,
                    format=plsc.PackFormat.INTERLEAVED)
            return c

        lax.fori_loop(0, 16, rbody, 0)

        t = s // _G
        g = s % _G
        orow = (n * _G + g) * _L + l0 + t * 16

        @pl.when(p == 0)
        def _():
            pltpu.async_copy(
                outbuf.at[pl.ds(0, 16)], out_hbm.at[pl.ds(orow, 16)], os0)

        @pl.when(p == 1)
        def _():
            pltpu.async_copy(
                outbuf.at[pl.ds(16, 16)], out_hbm.at[pl.ds(orow, 16)], os1)

        return carry

    lax.fori_loop(0, _NSTEP, body, 0)
    pltpu.make_async_copy(
        outbuf.at[pl.ds(0, 16)], out_hbm.at[pl.ds(0, 16)], os0).wait()
    pltpu.make_async_copy(
        outbuf.at[pl.ds(16, 16)], out_hbm.at[pl.ds(0, 16)], os1).wait()


def _sc_gather(table, om):
    mesh = plsc.VectorSubcoreMesh(core_axis_name="c", subcore_axis_name="s")
    f = functools.partial(
        pl.kernel,
        out_type=jax.ShapeDtypeStruct((_N * _G * _L, _GC), jnp.bfloat16),
        mesh=mesh,
        compiler_params=pltpu.CompilerParams(
            use_tc_tiling_on_sc=False, needs_layout_passes=False),
        scratch_types=[
            pltpu.VMEM((2 * _G * _K, _LW), jnp.float32),
            pltpu.VMEM((3, 2, _HB), jnp.int32),
            pltpu.VMEM((6 * _HB, _GC), jnp.bfloat16),
            pltpu.VMEM((32, _GC), jnp.bfloat16),
            pltpu.SemaphoreType.DMA,
            pltpu.SemaphoreType.DMA,
            pltpu.SemaphoreType.DMA,
            pltpu.SemaphoreType.DMA,
            pltpu.SemaphoreType.DMA,
        ],
    )(_sc_body)
    return f(table, om)


def _outproj_body(y, w, b, o_ref):
    g = pl.program_id(2)
    part = lax.dot_general(y[0, 0].astype(jnp.float32), w[0],
                           (((1,), (0,)), ((), ())),
                           preferred_element_type=jnp.float32)

    @pl.when(g == 0)
    def _():
        o_ref[0] = part + b[...]

    @pl.when(g > 0)
    def _():
        o_ref[0] += part


def kernel(x, dw_w, dw_b, ln_g, ln_b, off_w, off_b, mask_w, mask_b,
           in_w, in_b, out_w, out_b):
    n, l, c = x.shape
    dww = jnp.transpose(dw_w[:, 0, :])
    omw = jnp.concatenate([off_w, mask_w], axis=0)
    omb = jnp.concatenate([off_b, mask_b], axis=0)[:, None]

    proj, om = pl.pallas_call(
        _prep_body,
        grid=(_N, _NIP),
        in_specs=[
            pl.BlockSpec((1, _LBP, _C), lambda n_, i: (n_, jnp.maximum(i - 1, 0), 0)),
            pl.BlockSpec((1, _LBP, _C), lambda n_, i: (n_, i, 0)),
            pl.BlockSpec((1, _LBP, _C), lambda n_, i: (n_, jnp.minimum(i + 1, _NIP - 1), 0)),
            pl.BlockSpec((3, _C), lambda n_, i: (0, 0)),
            pl.BlockSpec((1, _C), lambda n_, i: (0, 0)),
            pl.BlockSpec((1, _C), lambda n_, i: (0, 0)),
            pl.BlockSpec((1, _C), lambda n_, i: (0, 0)),
            pl.BlockSpec((_C, _C), lambda n_, i: (0, 0)),
            pl.BlockSpec((1, _C), lambda n_, i: (0, 0)),
            pl.BlockSpec((_G * _K * 2, _C), lambda n_, i: (0, 0)),
            pl.BlockSpec((_G * _K * 2, 1), lambda n_, i: (0, 0)),
        ],
        out_specs=[
            pl.BlockSpec((1, _G, _LBP, _GC), lambda n_, i: (n_, 0, i, 0)),
            pl.BlockSpec((1, _G * _K * 2, _LBP), lambda n_, i: (n_, 0, i)),
        ],
        out_shape=[
            jax.ShapeDtypeStruct((_N, _G, _L, _GC), jnp.bfloat16),
            jax.ShapeDtypeStruct((_N, _G * _K * 2, _L), jnp.float32),
        ],
    )(x, x, x, dww, dw_b[None], ln_g[None], ln_b[None], in_w, in_b[None],
      omw, omb)

    y = _sc_gather(proj.reshape(_N * _G * _L, _GC), om)
    y = y.reshape(_N, _G, _L, _GC)

    w_r = jnp.transpose(out_w).reshape(_G, _GC, _C)
    o = pl.pallas_call(
        _outproj_body,
        grid=(_N, _NI, _G),
        in_specs=[
            pl.BlockSpec((1, 1, _LB, _GC), lambda n_, i, g: (n_, g, i, 0)),
            pl.BlockSpec((1, _GC, _C), lambda n_, i, g: (g, 0, 0)),
            pl.BlockSpec((1, _C), lambda n_, i, g: (0, 0)),
        ],
        out_specs=pl.BlockSpec((1, _LB, _C), lambda n_, i, g: (n_, i, 0)),
        out_shape=jax.ShapeDtypeStruct((_N, _L, _C), jnp.float32),
    )(y, w_r, out_b[None])
    return o
